# baseline (speedup 1.0000x reference)
import functools
import math

import jax
import jax.numpy as jnp
from jax import lax
from jax.experimental import pallas as pl
from jax.experimental.pallas import tpu as pltpu


def _fused_kernel(q_ref, k_ref, v_ref, wq_ref, wk_ref, wv_ref,
                  lna_ref, lnb_ref, o_ref, mk_sc, mv_sc,
                  *, eps, scale, n_p, bkv, n_sub):
    i = pl.program_id(0)

    @pl.when(i < n_p)
    def _project_kv():
        mk_sc[:, pl.ds(i * bkv, bkv)] = lax.dot_general(
            wk_ref[...].astype(jnp.bfloat16), k_ref[...].astype(jnp.bfloat16),
            (((0,), (1,)), ((), ())),
            preferred_element_type=jnp.float32).astype(jnp.bfloat16)
        mv_sc[pl.ds(i * bkv, bkv), :] = jnp.dot(
            v_ref[...].astype(jnp.bfloat16), wv_ref[...].astype(jnp.bfloat16),
            preferred_element_type=jnp.float32).astype(jnp.bfloat16)

    @pl.when(i >= n_p)
    def _attend():
        bq, d = q_ref.shape
        sub = bq // n_sub
        log2e = 1.4426950408889634
        for h in range(n_sub):
            qf = q_ref[pl.ds(h * sub, sub), :]
            mq = (jnp.dot(qf.astype(jnp.bfloat16),
                          wq_ref[...].astype(jnp.bfloat16),
                          preferred_element_type=jnp.float32)
                  * (scale * log2e)).astype(jnp.bfloat16)
            s = jnp.dot(mq, mk_sc[...],
                        preferred_element_type=jnp.float32)
            m = jnp.max(s, axis=-1, keepdims=True)
            p = jnp.exp2(s - m)
            l = jnp.sum(p, axis=-1, keepdims=True)
            o = jnp.dot(p.astype(jnp.bfloat16), mv_sc[...],
                        preferred_element_type=jnp.float32)
            z = o / l + qf
            mu = jnp.mean(z, axis=-1, keepdims=True)
            sigma = jnp.sqrt(
                jnp.sum((z - mu) ** 2, axis=-1, keepdims=True)
                * (1.0 / (d - 1)))
            o_ref[pl.ds(h * sub, sub), :] = (
                (z - mu) / (sigma + eps) * lna_ref[...]
                + lnb_ref[...]).astype(o_ref.dtype)


def kernel(q, k, v, w_qs, w_ks, w_vs, ln_a, ln_b):
    eps = 1e-3
    L, D = q.shape
    kd = w_qs.shape[1]
    bq = min(2048, L)
    n_sub = bq // 512 if bq >= 1024 else 1
    bkv = min(2048, L)
    n_q = L // bq
    n_p = L // bkv

    lna = jnp.reshape(ln_a, (1, D)).astype(jnp.float32)
    lnb = jnp.reshape(ln_b, (1, D)).astype(jnp.float32)

    def kv_idx(i):
        return (jnp.minimum(i, n_p - 1), 0)

    def q_idx(i):
        return (jnp.maximum(i - n_p, 0), 0)

    return pl.pallas_call(
        functools.partial(_fused_kernel, eps=eps, scale=1.0 / math.sqrt(D),
                          n_p=n_p, bkv=bkv, n_sub=n_sub),
        grid=(n_p + n_q,),
        in_specs=[
            pl.BlockSpec((bq, D), q_idx),
            pl.BlockSpec((bkv, D), kv_idx),
            pl.BlockSpec((bkv, D), kv_idx),
            pl.BlockSpec((D, kd), lambda i: (0, 0)),
            pl.BlockSpec((D, kd), lambda i: (0, 0)),
            pl.BlockSpec((D, kd), lambda i: (0, 0)),
            pl.BlockSpec((1, D), lambda i: (0, 0)),
            pl.BlockSpec((1, D), lambda i: (0, 0)),
        ],
        out_specs=pl.BlockSpec((bq, D), q_idx),
        out_shape=jax.ShapeDtypeStruct((L, D), jnp.float32),
        scratch_shapes=[
            pltpu.VMEM((kd, L), jnp.bfloat16),
            pltpu.VMEM((L, kd), jnp.bfloat16),
        ],
        compiler_params=pltpu.CompilerParams(
            dimension_semantics=("arbitrary",),
            vmem_limit_bytes=100 * 1024 * 1024,
        ),
    )(q, k, v, w_qs, w_ks, w_vs, lna, lnb)


# speedup vs baseline: 1.0009x; 1.0009x over previous
import functools
import math

import jax
import jax.numpy as jnp
from jax import lax
from jax.experimental import pallas as pl
from jax.experimental.pallas import tpu as pltpu


def _fused_kernel(q_ref, k_ref, v_ref, wq_ref, wk_ref, wv_ref,
                  lna_ref, lnb_ref, o_ref, mk_sc, mv_sc,
                  *, eps, scale, n_p, bkv, n_sub):
    i = pl.program_id(0)

    @pl.when(i < n_p)
    def _project_kv():
        mk_sc[:, pl.ds(i * bkv, bkv)] = lax.dot_general(
            wk_ref[...].astype(jnp.bfloat16), k_ref[...].astype(jnp.bfloat16),
            (((0,), (1,)), ((), ())),
            preferred_element_type=jnp.float32).astype(jnp.bfloat16)
        mv_sc[pl.ds(i * bkv, bkv), :] = jnp.dot(
            v_ref[...].astype(jnp.bfloat16), wv_ref[...].astype(jnp.bfloat16),
            preferred_element_type=jnp.float32).astype(jnp.bfloat16)

    @pl.when(i >= n_p)
    def _attend():
        bq, d = q_ref.shape
        sub = bq // n_sub
        log2e = 1.4426950408889634
        for h in range(n_sub):
            qf = q_ref[pl.ds(h * sub, sub), :]
            mq = (jnp.dot(qf.astype(jnp.bfloat16),
                          wq_ref[...].astype(jnp.bfloat16),
                          preferred_element_type=jnp.float32)
                  * (scale * log2e)).astype(jnp.bfloat16)
            s = jnp.dot(mq, mk_sc[...],
                        preferred_element_type=jnp.float32)
            m = jnp.max(s, axis=-1, keepdims=True)
            p = jnp.exp2(s - m)
            l = jnp.sum(p, axis=-1, keepdims=True)
            o = jnp.dot(p.astype(jnp.bfloat16), mv_sc[...],
                        preferred_element_type=jnp.float32)
            z = o / l + qf
            mu = jnp.mean(z, axis=-1, keepdims=True)
            sigma = jnp.sqrt(
                jnp.sum((z - mu) ** 2, axis=-1, keepdims=True)
                * (1.0 / (d - 1)))
            o_ref[pl.ds(h * sub, sub), :] = (
                (z - mu) / (sigma + eps) * lna_ref[...]
                + lnb_ref[...]).astype(o_ref.dtype)


def kernel(q, k, v, w_qs, w_ks, w_vs, ln_a, ln_b):
    eps = 1e-3
    L, D = q.shape
    kd = w_qs.shape[1]
    bq = min(2048, L)
    n_sub = bq // 512 if bq >= 1024 else 1
    bkv = min(1024, L)
    n_q = L // bq
    n_p = L // bkv

    lna = jnp.reshape(ln_a, (1, D)).astype(jnp.float32)
    lnb = jnp.reshape(ln_b, (1, D)).astype(jnp.float32)

    def kv_idx(i):
        return (jnp.minimum(i, n_p - 1), 0)

    def q_idx(i):
        return (jnp.maximum(i - n_p, 0), 0)

    return pl.pallas_call(
        functools.partial(_fused_kernel, eps=eps, scale=1.0 / math.sqrt(D),
                          n_p=n_p, bkv=bkv, n_sub=n_sub),
        grid=(n_p + n_q,),
        in_specs=[
            pl.BlockSpec((bq, D), q_idx),
            pl.BlockSpec((bkv, D), kv_idx),
            pl.BlockSpec((bkv, D), kv_idx),
            pl.BlockSpec((D, kd), lambda i: (0, 0)),
            pl.BlockSpec((D, kd), lambda i: (0, 0)),
            pl.BlockSpec((D, kd), lambda i: (0, 0)),
            pl.BlockSpec((1, D), lambda i: (0, 0)),
            pl.BlockSpec((1, D), lambda i: (0, 0)),
        ],
        out_specs=pl.BlockSpec((bq, D), q_idx),
        out_shape=jax.ShapeDtypeStruct((L, D), jnp.float32),
        scratch_shapes=[
            pltpu.VMEM((kd, L), jnp.bfloat16),
            pltpu.VMEM((L, kd), jnp.bfloat16),
        ],
        compiler_params=pltpu.CompilerParams(
            dimension_semantics=("arbitrary",),
            vmem_limit_bytes=100 * 1024 * 1024,
        ),
    )(q, k, v, w_qs, w_ks, w_vs, lna, lnb)
